# baseline (speedup 1.0000x reference)
import functools

import jax
import jax.numpy as jnp
import numpy as np
from jax import lax
from jax.experimental import pallas as pl
from jax.experimental.pallas import tpu as pltpu
from jax.experimental.pallas import tpu_sc as plsc

_N_NODES = 10000
_N_EDGES = 160000
_D = 256


_ROW_BLK = 1000


def _node_mm_body(x_ref, w_ref, b_ref, y_ref):
    w = w_ref[...]
    ws = (w[:_D, :] + w[_D:, :]).astype(jnp.bfloat16)
    xb = x_ref[...].astype(jnp.bfloat16)
    y = jnp.dot(xb, ws, preferred_element_type=jnp.float32) + b_ref[...]
    bits = lax.bitcast_convert_type(y, jnp.uint32)
    rnd = bits + jnp.uint32(0x7FFF) + ((bits >> 16) & jnp.uint32(1))
    r4 = rnd.reshape(_ROW_BLK, _D // 32, 2, 16)
    lo = (r4[:, :, 0, :].reshape(_ROW_BLK, _D // 2) >> 16) & jnp.uint32(0xFFFF)
    hi = r4[:, :, 1, :].reshape(_ROW_BLK, _D // 2) & jnp.uint32(0xFFFF0000)
    y_ref[...] = lax.bitcast_convert_type(lo | hi, jnp.int32)


def _node_transform(x, w, b2):
    grid = (_N_NODES // _ROW_BLK,)
    return pl.pallas_call(
        _node_mm_body,
        grid=grid,
        in_specs=[
            pl.BlockSpec((_ROW_BLK, _D), lambda i: (i, 0)),
            pl.BlockSpec((2 * _D, _D), lambda i: (0, 0)),
            pl.BlockSpec((1, _D), lambda i: (0, 0)),
        ],
        out_specs=pl.BlockSpec((_ROW_BLK, _D // 2), lambda i: (i, 0)),
        out_shape=jax.ShapeDtypeStruct((_N_NODES, _D // 2), jnp.int32),
    )(x, w, b2)


_NC = 2
_NS = 16
_NW = _NC * _NS
_EPW = _N_EDGES // _NW
_CHUNK = 40
_PH0 = 2520
_PH1 = _EPW - _PH0


def _edge_body(y_hbm, ei_hbm, out_hbm, y_sh, src_v, dst_v,
               bufa0, bufb0, bufa1, bufb1, outv0, outv1,
               sa0, sb0, sa1, sb1, sw0, sw1):
    wid = lax.axis_index("s") * _NC + lax.axis_index("c")
    base = wid * _EPW

    sid = lax.axis_index("s")
    t_base = sid * 632

    @pl.when(sid < 15)
    def _():
        pltpu.sync_copy(y_hbm.at[pl.ds(t_base, 632)],
                        y_sh.at[pl.ds(t_base, 632)])

    @pl.when(sid == 15)
    def _():
        pltpu.sync_copy(y_hbm.at[pl.ds(15 * 632, _N_NODES - 15 * 632)],
                        y_sh.at[pl.ds(15 * 632, _N_NODES - 15 * 632)])

    plsc.subcore_barrier()

    bufs = ((bufa0, bufb0, outv0, sa0, sb0, sw0), (bufa1, bufb1, outv1, sa1, sb1, sw1))

    def run_phase(ph_base, ph_n, first_phase):
        nchunk = ph_n // _CHUNK
        pltpu.sync_copy(ei_hbm.at[pl.ds(ph_base, ph_n)],
                        src_v.at[pl.ds(0, ph_n)])
        pltpu.sync_copy(ei_hbm.at[pl.ds(_N_EDGES + ph_base, ph_n)],
                        dst_v.at[pl.ds(0, ph_n)])

        def issue(c, slot):
            ba, bb, _, sa, sb, _ = bufs[slot]
            off = c * _CHUNK
            pltpu.async_copy(y_sh.at[src_v.at[pl.ds(off, _CHUNK)]], ba, sa)
            pltpu.async_copy(y_sh.at[dst_v.at[pl.ds(off, _CHUNK)]], bb, sb)

        def process(c, slot):
            ba, bb, ov, sa, sb, sw = bufs[slot]
            pltpu.make_async_copy(y_sh.at[src_v.at[pl.ds(0, _CHUNK)]], ba, sa).wait()
            pltpu.make_async_copy(y_sh.at[dst_v.at[pl.ds(0, _CHUNK)]], bb, sb).wait()

            not_first = jnp.logical_or(c >= 2, jnp.bool_(not first_phase))

            @pl.when(not_first)
            def _():
                pltpu.make_async_copy(ov, out_hbm.at[pl.ds(base, _CHUNK)], sw).wait()

            @plsc.parallel_loop(0, _CHUNK, unroll=2)
            def row_body(r):
                for j in range(_D // 32):
                    sl = pl.ds(j * 16, 16)
                    wa = ba[r, sl]
                    wb = bb[r, sl]
                    ov[r, pl.ds(j * 32, 16)] = (
                        lax.bitcast_convert_type(wa << 16, jnp.float32)
                        + lax.bitcast_convert_type(wb << 16, jnp.float32)
                    )
                    ov[r, pl.ds(j * 32 + 16, 16)] = (
                        lax.bitcast_convert_type(wa & -65536, jnp.float32)
                        + lax.bitcast_convert_type(wb & -65536, jnp.float32)
                    )
            pltpu.async_copy(
                ov, out_hbm.at[pl.ds(ph_base + c * _CHUNK, _CHUNK)], sw)

        issue(0, 0)

        def pair_body(g, carry):
            issue(2 * g + 1, 1)
            process(2 * g, 0)
            issue(2 * g + 2, 0)
            process(2 * g + 1, 1)
            return carry

        lax.fori_loop(0, (nchunk - 1) // 2, pair_body, 0)
        if nchunk % 2 == 1:
            process(nchunk - 1, 0)
        else:
            issue(nchunk - 1, 1)
            process(nchunk - 2, 0)
            process(nchunk - 1, 1)

    run_phase(base, _PH0, True)
    run_phase(base + _PH0, _PH1, False)
    pltpu.make_async_copy(outv0, out_hbm.at[pl.ds(base, _CHUNK)], sw0).wait()
    pltpu.make_async_copy(outv1, out_hbm.at[pl.ds(base, _CHUNK)], sw1).wait()


@functools.partial(
    pl.kernel,
    out_type=jax.ShapeDtypeStruct((_N_EDGES, _D), jnp.float32),
    mesh=plsc.VectorSubcoreMesh(core_axis_name="c", subcore_axis_name="s"),
    scratch_types=[
        pltpu.VMEM_SHARED((_N_NODES, _D // 2), jnp.int32),
        pltpu.VMEM((_PH0,), jnp.int32),
        pltpu.VMEM((_PH0,), jnp.int32),
        pltpu.VMEM((_CHUNK, _D // 2), jnp.int32),
        pltpu.VMEM((_CHUNK, _D // 2), jnp.int32),
        pltpu.VMEM((_CHUNK, _D // 2), jnp.int32),
        pltpu.VMEM((_CHUNK, _D // 2), jnp.int32),
        pltpu.VMEM((_CHUNK, _D), jnp.float32),
        pltpu.VMEM((_CHUNK, _D), jnp.float32),
        pltpu.SemaphoreType.DMA,
        pltpu.SemaphoreType.DMA,
        pltpu.SemaphoreType.DMA,
        pltpu.SemaphoreType.DMA,
        pltpu.SemaphoreType.DMA,
        pltpu.SemaphoreType.DMA,
    ],
)
def _edge_gather_add(y_hbm, ei_hbm, out_hbm, y_sh, src_v, dst_v,
                     bufa0, bufb0, bufa1, bufb1, outv0, outv1,
                     sa0, sb0, sa1, sb1, sw0, sw1):
    _edge_body(y_hbm, ei_hbm, out_hbm, y_sh, src_v, dst_v,
               bufa0, bufb0, bufa1, bufb1, outv0, outv1,
               sa0, sb0, sa1, sb1, sw0, sw1)


def kernel(x, edge_index, W, b):
    ei = edge_index.astype(jnp.int32).reshape(2 * _N_EDGES)
    y = _node_transform(x, W, b.reshape(1, _D))
    return _edge_gather_add(y, ei)


# speedup vs baseline: 1.5324x; 1.5324x over previous
import functools

import jax
import jax.numpy as jnp
import numpy as np
from jax import lax
from jax.experimental import pallas as pl
from jax.experimental.pallas import tpu as pltpu
from jax.experimental.pallas import tpu_sc as plsc

_N_NODES = 10000
_N_EDGES = 160000
_D = 256

_K = np.arange(_D // 2)
_PERM_LO = 32 * (_K // 16) + (_K % 16)
_PERM_HI = _PERM_LO + 16
_PERM = np.concatenate([_PERM_LO, _PERM_HI])


_ROW_BLK = 1000


def _node_mm_body(x_ref, w_ref, b_ref, y_ref):
    w = w_ref[...]
    ws = w[:_D, :] + w[_D:, :]
    y = jnp.dot(x_ref[...], ws, preferred_element_type=jnp.float32) + b_ref[...]
    bits = lax.bitcast_convert_type(y, jnp.uint32)
    rnd = bits + jnp.uint32(0x7FFF) + ((bits >> 16) & jnp.uint32(1))
    lo = (rnd[:, : _D // 2] >> 16) & jnp.uint32(0xFFFF)
    hi = rnd[:, _D // 2 :] & jnp.uint32(0xFFFF0000)
    y_ref[...] = lax.bitcast_convert_type(lo | hi, jnp.int32)


def _node_transform(x, w, b2):
    grid = (_N_NODES // _ROW_BLK,)
    return pl.pallas_call(
        _node_mm_body,
        grid=grid,
        in_specs=[
            pl.BlockSpec((_ROW_BLK, _D), lambda i: (i, 0)),
            pl.BlockSpec((2 * _D, _D), lambda i: (0, 0)),
            pl.BlockSpec((1, _D), lambda i: (0, 0)),
        ],
        out_specs=pl.BlockSpec((_ROW_BLK, _D // 2), lambda i: (i, 0)),
        out_shape=jax.ShapeDtypeStruct((_N_NODES, _D // 2), jnp.int32),
    )(x, w, b2)


_NC = 2
_NS = 16
_NW = _NC * _NS
_EPW = _N_EDGES // _NW
_CHUNK = 40
_PH0 = 2520
_PH1 = _EPW - _PH0


def _edge_body(y_hbm, ei_hbm, out_hbm, y_sh, src_v, dst_v,
               bufa0, bufb0, bufa1, bufb1, outv0, outv1,
               sa0, sb0, sa1, sb1, sw0, sw1):
    wid = lax.axis_index("s") * _NC + lax.axis_index("c")
    base = wid * _EPW

    sid = lax.axis_index("s")
    t_base = sid * 632

    @pl.when(sid < 15)
    def _():
        pltpu.sync_copy(y_hbm.at[pl.ds(t_base, 632)],
                        y_sh.at[pl.ds(t_base, 632)])

    @pl.when(sid == 15)
    def _():
        pltpu.sync_copy(y_hbm.at[pl.ds(15 * 632, _N_NODES - 15 * 632)],
                        y_sh.at[pl.ds(15 * 632, _N_NODES - 15 * 632)])

    plsc.subcore_barrier()

    bufs = ((bufa0, bufb0, outv0, sa0, sb0, sw0), (bufa1, bufb1, outv1, sa1, sb1, sw1))

    def run_phase(ph_base, ph_n, first_phase):
        nchunk = ph_n // _CHUNK
        pltpu.sync_copy(ei_hbm.at[pl.ds(ph_base, ph_n)],
                        src_v.at[pl.ds(0, ph_n)])
        pltpu.sync_copy(ei_hbm.at[pl.ds(_N_EDGES + ph_base, ph_n)],
                        dst_v.at[pl.ds(0, ph_n)])

        def issue(c, slot):
            ba, bb, _, sa, sb, _ = bufs[slot]
            off = c * _CHUNK
            pltpu.async_copy(y_sh.at[src_v.at[pl.ds(off, _CHUNK)]], ba, sa)
            pltpu.async_copy(y_sh.at[dst_v.at[pl.ds(off, _CHUNK)]], bb, sb)

        def process(c, slot):
            ba, bb, ov, sa, sb, sw = bufs[slot]
            pltpu.make_async_copy(y_sh.at[src_v.at[pl.ds(0, _CHUNK)]], ba, sa).wait()
            pltpu.make_async_copy(y_sh.at[dst_v.at[pl.ds(0, _CHUNK)]], bb, sb).wait()

            not_first = jnp.logical_or(c >= 2, jnp.bool_(not first_phase))

            @pl.when(not_first)
            def _():
                pltpu.make_async_copy(ov, out_hbm.at[pl.ds(base, _CHUNK)], sw).wait()

            @plsc.parallel_loop(0, _CHUNK, unroll=2)
            def row_body(r):
                for j in range(_D // 32):
                    sl = pl.ds(j * 16, 16)
                    wa = ba[r, sl]
                    wb = bb[r, sl]
                    ov[r, pl.ds(j * 32, 16)] = (
                        lax.bitcast_convert_type(wa << 16, jnp.float32)
                        + lax.bitcast_convert_type(wb << 16, jnp.float32)
                    )
                    ov[r, pl.ds(j * 32 + 16, 16)] = (
                        lax.bitcast_convert_type(wa & -65536, jnp.float32)
                        + lax.bitcast_convert_type(wb & -65536, jnp.float32)
                    )
            pltpu.async_copy(
                ov, out_hbm.at[pl.ds(ph_base + c * _CHUNK, _CHUNK)], sw)

        issue(0, 0)

        def pair_body(g, carry):
            issue(2 * g + 1, 1)
            process(2 * g, 0)
            issue(2 * g + 2, 0)
            process(2 * g + 1, 1)
            return carry

        lax.fori_loop(0, (nchunk - 1) // 2, pair_body, 0)
        if nchunk % 2 == 1:
            process(nchunk - 1, 0)
        else:
            issue(nchunk - 1, 1)
            process(nchunk - 2, 0)
            process(nchunk - 1, 1)

    run_phase(base, _PH0, True)
    run_phase(base + _PH0, _PH1, False)
    pltpu.make_async_copy(outv0, out_hbm.at[pl.ds(base, _CHUNK)], sw0).wait()
    pltpu.make_async_copy(outv1, out_hbm.at[pl.ds(base, _CHUNK)], sw1).wait()


@functools.partial(
    pl.kernel,
    out_type=jax.ShapeDtypeStruct((_N_EDGES, _D), jnp.float32),
    mesh=plsc.VectorSubcoreMesh(core_axis_name="c", subcore_axis_name="s"),
    scratch_types=[
        pltpu.VMEM_SHARED((_N_NODES, _D // 2), jnp.int32),
        pltpu.VMEM((_PH0,), jnp.int32),
        pltpu.VMEM((_PH0,), jnp.int32),
        pltpu.VMEM((_CHUNK, _D // 2), jnp.int32),
        pltpu.VMEM((_CHUNK, _D // 2), jnp.int32),
        pltpu.VMEM((_CHUNK, _D // 2), jnp.int32),
        pltpu.VMEM((_CHUNK, _D // 2), jnp.int32),
        pltpu.VMEM((_CHUNK, _D), jnp.float32),
        pltpu.VMEM((_CHUNK, _D), jnp.float32),
        pltpu.SemaphoreType.DMA,
        pltpu.SemaphoreType.DMA,
        pltpu.SemaphoreType.DMA,
        pltpu.SemaphoreType.DMA,
        pltpu.SemaphoreType.DMA,
        pltpu.SemaphoreType.DMA,
    ],
)
def _edge_gather_add(y_hbm, ei_hbm, out_hbm, y_sh, src_v, dst_v,
                     bufa0, bufb0, bufa1, bufb1, outv0, outv1,
                     sa0, sb0, sa1, sb1, sw0, sw1):
    _edge_body(y_hbm, ei_hbm, out_hbm, y_sh, src_v, dst_v,
               bufa0, bufb0, bufa1, bufb1, outv0, outv1,
               sa0, sb0, sa1, sb1, sw0, sw1)


def kernel(x, edge_index, W, b):
    ei = edge_index.astype(jnp.int32).reshape(2 * _N_EDGES)
    perm = jnp.asarray(_PERM)
    w_sw = W[:, perm]
    b_sw = b[perm].reshape(1, _D)
    y = _node_transform(x, w_sw, b_sw)
    return _edge_gather_add(y, ei)
